# Initial kernel scaffold; baseline (speedup 1.0000x reference)
import jax
import jax.numpy as jnp
from jax.experimental import pallas as pl


def kernel(features, edge_idx, W1, b1, W2, b2):
    raise NotImplementedError("write your pallas kernel here")


# speedup vs baseline: 19.7527x; 19.7527x over previous
import functools

import jax
import jax.numpy as jnp
from jax import lax
from jax.experimental import pallas as pl
from jax.experimental.pallas import tpu as pltpu
from jax.experimental.pallas import tpu_sc as plsc

N = 10000
E = 320000
D = 128
F1 = 64
C = 40
F2 = 48

NP = 10240
NC = 2
NS = 16
NW = NC * NS
CH = 128
NCH = 79
EPT = NCH * CH
EPAD = NW * EPT
SLC = NP // NS

BR = 512
GRID = NP // BR

_MESH = plsc.VectorSubcoreMesh(core_axis_name="c", subcore_axis_name="s")


@functools.partial(
    pl.kernel,
    out_type=jax.ShapeDtypeStruct((NC, NP), jnp.float32),
    mesh=_MESH,
    scratch_types=[
        pltpu.VMEM((NCH, CH), jnp.int32),
        pltpu.VMEM((CH,), jnp.float32),
        pltpu.VMEM_SHARED((NP,), jnp.float32),
    ],
)
def _sc_degree(col_hbm, zero1_hbm, out_hbm, colv, ones_v, accum):
    cid = lax.axis_index("c")
    sid = lax.axis_index("s")
    wid = sid * NC + cid
    for i in range(CH // 16):
        ones_v[pl.ds(i * 16, 16)] = jnp.ones((16,), jnp.float32)
    pltpu.sync_copy(zero1_hbm.at[pl.ds(sid * SLC, SLC)],
                    accum.at[pl.ds(sid * SLC, SLC)])
    pltpu.sync_copy(col_hbm.at[wid], colv)
    plsc.subcore_barrier()

    def body(ch, carry):
        pltpu.sync_copy(ones_v, accum.at[colv.at[ch]], add=True)
        return carry

    lax.fori_loop(0, NCH, body, 0)
    plsc.subcore_barrier()
    pltpu.sync_copy(accum.at[pl.ds(sid * SLC, SLC)],
                    out_hbm.at[cid, pl.ds(sid * SLC, SLC)])


def _make_sc_gather_add(F):
    @functools.partial(
        pl.kernel,
        out_type=jax.ShapeDtypeStruct((NC, NP, F), jnp.float32),
        mesh=_MESH,
        compiler_params=pltpu.CompilerParams(use_tc_tiling_on_sc=False),
        scratch_types=[
            pltpu.VMEM((NCH, CH), jnp.int32),
            pltpu.VMEM((NCH, CH), jnp.int32),
            pltpu.VMEM((CH, F), jnp.float32),
            pltpu.VMEM_SHARED((NP, F), jnp.float32),
            pltpu.SemaphoreType.DMA,
        ],
    )
    def sc_gather_add(table_hbm, row_hbm, col_hbm, zero2_hbm, out_hbm,
                      rowv, colv, rows_v, accum, sem):
        cid = lax.axis_index("c")
        sid = lax.axis_index("s")
        wid = sid * NC + cid
        pltpu.sync_copy(zero2_hbm.at[pl.ds(sid * SLC, SLC)],
                        accum.at[pl.ds(sid * SLC, SLC)])
        pltpu.sync_copy(row_hbm.at[wid], rowv)
        pltpu.sync_copy(col_hbm.at[wid], colv)
        plsc.subcore_barrier()

        def body(ch, carry):
            pltpu.async_copy(table_hbm.at[rowv.at[ch]], rows_v, sem).wait()
            pltpu.sync_copy(rows_v, accum.at[colv.at[ch]], add=True)
            return carry

        lax.fori_loop(0, NCH, body, 0)
        plsc.subcore_barrier()
        pltpu.sync_copy(accum.at[pl.ds(sid * SLC, SLC)],
                        out_hbm.at[cid, pl.ds(sid * SLC, SLC)])

    return sc_gather_add


_sc_gather_add_f1 = _make_sc_gather_add(F1)
_sc_gather_add_f2 = _make_sc_gather_add(F2)


def _dis(deg_ref):
    return lax.rsqrt(deg_ref[0] + deg_ref[1] + 1.0)[:, None]


def _t1_body(deg_ref, x_ref, w_ref, o_ref):
    m = jnp.dot(x_ref[...], w_ref[...], preferred_element_type=jnp.float32)
    o_ref[...] = m * _dis(deg_ref)


def _t2_body(deg_ref, p_ref, h_ref, b_ref, w_ref, o_ref):
    dis = _dis(deg_ref)
    x1 = jnp.maximum((p_ref[0] + p_ref[1] + h_ref[...]) * dis + b_ref[...], 0.0)
    o_ref[...] = jnp.dot(x1, w_ref[...], preferred_element_type=jnp.float32) * dis


def _t3_body(deg_ref, p_ref, h_ref, b_ref, o_ref):
    o_ref[...] = (p_ref[0] + p_ref[1] + h_ref[...]) * _dis(deg_ref) + b_ref[...]


def _tc_h1(degp, xp, w1):
    return pl.pallas_call(
        _t1_body,
        grid=(GRID,),
        in_specs=[
            pl.BlockSpec((NC, BR), lambda i: (0, i)),
            pl.BlockSpec((BR, D), lambda i: (i, 0)),
            pl.BlockSpec((D, F1), lambda i: (0, 0)),
        ],
        out_specs=pl.BlockSpec((BR, F1), lambda i: (i, 0)),
        out_shape=jax.ShapeDtypeStruct((NP, F1), jnp.float32),
    )(degp, xp, w1)


def _tc_h2(degp, p1, h1, b1, w2p):
    return pl.pallas_call(
        _t2_body,
        grid=(GRID,),
        in_specs=[
            pl.BlockSpec((NC, BR), lambda i: (0, i)),
            pl.BlockSpec((NC, BR, F1), lambda i: (0, i, 0)),
            pl.BlockSpec((BR, F1), lambda i: (i, 0)),
            pl.BlockSpec((1, F1), lambda i: (0, 0)),
            pl.BlockSpec((F1, F2), lambda i: (0, 0)),
        ],
        out_specs=pl.BlockSpec((BR, F2), lambda i: (i, 0)),
        out_shape=jax.ShapeDtypeStruct((NP, F2), jnp.float32),
    )(degp, p1, h1, b1, w2p)


def _tc_out(degp, p2, h2, b2p):
    return pl.pallas_call(
        _t3_body,
        grid=(GRID,),
        in_specs=[
            pl.BlockSpec((NC, BR), lambda i: (0, i)),
            pl.BlockSpec((NC, BR, F2), lambda i: (0, i, 0)),
            pl.BlockSpec((BR, F2), lambda i: (i, 0)),
            pl.BlockSpec((1, F2), lambda i: (0, 0)),
        ],
        out_specs=pl.BlockSpec((BR, F2), lambda i: (i, 0)),
        out_shape=jax.ShapeDtypeStruct((NP, F2), jnp.float32),
    )(degp, p2, h2, b2p)


def kernel(features, edge_idx, W1, b1, W2, b2):
    row = edge_idx[0]
    col = edge_idx[1]
    pad = EPAD - E
    rowp = jnp.concatenate(
        [row, jnp.zeros((pad,), jnp.int32)]).reshape(NW, NCH, CH)
    colp = jnp.concatenate(
        [col, jnp.full((pad,), N, jnp.int32)]).reshape(NW, NCH, CH)

    xp = jnp.pad(features, ((0, NP - N), (0, 0)))
    w2p = jnp.pad(W2, ((0, 0), (0, F2 - C)))
    b1r = b1.reshape(1, F1)
    b2r = jnp.pad(b2, (0, F2 - C)).reshape(1, F2)
    z1 = jnp.zeros((NP,), jnp.float32)
    z64 = jnp.zeros((NP, F1), jnp.float32)
    z48 = jnp.zeros((NP, F2), jnp.float32)

    degp = _sc_degree(colp, z1)
    h1 = _tc_h1(degp, xp, W1)
    p1 = _sc_gather_add_f1(h1, rowp, colp, z64)
    h2 = _tc_h2(degp, p1, h1, b1r, w2p)
    p2 = _sc_gather_add_f2(h2, rowp, colp, z48)
    out = _tc_out(degp, p2, h2, b2r)
    return out[:N, :C]


# speedup vs baseline: 25.3512x; 1.2834x over previous
import functools

import jax
import jax.numpy as jnp
from jax import lax
from jax.experimental import pallas as pl
from jax.experimental.pallas import tpu as pltpu
from jax.experimental.pallas import tpu_sc as plsc

N = 10000
E = 320000
D = 128
F1 = 64
C = 40
F2 = 48

NP = 10240
NC = 2
NS = 16
NW = NC * NS
CH = 128
NCH = 79
EPT = NCH * CH
EPAD = NW * EPT
SLC = NP // NS
NG = 4
NSC = 4
NBUF = NG + NSC

BR = 512
GRID = NP // BR

_MESH = plsc.VectorSubcoreMesh(core_axis_name="c", subcore_axis_name="s")


@functools.partial(
    pl.kernel,
    out_type=jax.ShapeDtypeStruct((NC, NP), jnp.float32),
    mesh=_MESH,
    scratch_types=[
        pltpu.VMEM((NCH, CH), jnp.int32),
        pltpu.VMEM((CH,), jnp.float32),
        pltpu.VMEM_SHARED((NP,), jnp.float32),
    ],
)
def _sc_degree(col_hbm, zero1_hbm, out_hbm, colv, ones_v, accum):
    cid = lax.axis_index("c")
    sid = lax.axis_index("s")
    wid = sid * NC + cid
    for i in range(CH // 16):
        ones_v[pl.ds(i * 16, 16)] = jnp.ones((16,), jnp.float32)
    pltpu.sync_copy(zero1_hbm.at[pl.ds(sid * SLC, SLC)],
                    accum.at[pl.ds(sid * SLC, SLC)])
    pltpu.sync_copy(col_hbm.at[wid], colv)
    plsc.subcore_barrier()

    def body(ch, carry):
        pltpu.sync_copy(ones_v, accum.at[colv.at[ch]], add=True)
        return carry

    lax.fori_loop(0, NCH, body, 0)
    plsc.subcore_barrier()
    pltpu.sync_copy(accum.at[pl.ds(sid * SLC, SLC)],
                    out_hbm.at[cid, pl.ds(sid * SLC, SLC)])


def _make_sc_gather_add(F):
    @functools.partial(
        pl.kernel,
        out_type=jax.ShapeDtypeStruct((NC, NP, F), jnp.float32),
        mesh=_MESH,
        compiler_params=pltpu.CompilerParams(use_tc_tiling_on_sc=False),
        scratch_types=[
            pltpu.VMEM((NCH, CH), jnp.int32),
            pltpu.VMEM((NCH, CH), jnp.int32),
            pltpu.VMEM((NBUF, CH, F), jnp.float32),
            pltpu.VMEM_SHARED((NP, F), jnp.float32),
            pltpu.SemaphoreType.DMA((NBUF,)),
            pltpu.SemaphoreType.DMA((NBUF,)),
        ],
    )
    def sc_gather_add(table_hbm, row_hbm, col_hbm, zero2_hbm, out_hbm,
                      rowv, colv, rows_v, accum, gsem, ssem):
        cid = lax.axis_index("c")
        sid = lax.axis_index("s")
        wid = sid * NC + cid
        pltpu.sync_copy(zero2_hbm.at[pl.ds(sid * SLC, SLC)],
                        accum.at[pl.ds(sid * SLC, SLC)])
        pltpu.sync_copy(row_hbm.at[wid], rowv)
        pltpu.sync_copy(col_hbm.at[wid], colv)
        plsc.subcore_barrier()

        for b in range(NG):
            pltpu.async_copy(table_hbm.at[rowv.at[b]], rows_v.at[b],
                             gsem.at[b])

        def body(ch, carry):
            buf = lax.rem(ch, NBUF)
            pltpu.make_async_copy(table_hbm.at[rowv.at[ch]],
                                  rows_v.at[buf], gsem.at[buf]).wait()
            pltpu.async_copy(rows_v.at[buf], accum.at[colv.at[ch]],
                             ssem.at[buf], add=True)
            prv = ch - NSC
            nxt = ch + NG

            @pl.when(nxt < NCH)
            def _():
                pbuf = lax.rem(nxt, NBUF)

                @pl.when(prv >= 0)
                def _():
                    pltpu.make_async_copy(rows_v.at[pbuf],
                                          accum.at[colv.at[ch]],
                                          ssem.at[pbuf]).wait()

                pltpu.async_copy(table_hbm.at[rowv.at[nxt]],
                                 rows_v.at[pbuf], gsem.at[pbuf])

            return carry

        lax.fori_loop(0, NCH, body, 0)
        for k in range(max(0, NCH - NSC - NG), NCH):
            pltpu.make_async_copy(rows_v.at[k % NBUF],
                                  accum.at[colv.at[0]],
                                  ssem.at[k % NBUF]).wait()
        plsc.subcore_barrier()
        pltpu.sync_copy(accum.at[pl.ds(sid * SLC, SLC)],
                        out_hbm.at[cid, pl.ds(sid * SLC, SLC)])

    return sc_gather_add


_sc_gather_add_f1 = _make_sc_gather_add(F1)
_sc_gather_add_f2 = _make_sc_gather_add(F2)


def _dis(deg_ref):
    return lax.rsqrt(deg_ref[0] + deg_ref[1] + 1.0)[:, None]


def _t1_body(deg_ref, x_ref, w_ref, o_ref):
    m = jnp.dot(x_ref[...], w_ref[...], preferred_element_type=jnp.float32)
    o_ref[...] = m * _dis(deg_ref)


def _t2_body(deg_ref, p_ref, h_ref, b_ref, w_ref, o_ref):
    dis = _dis(deg_ref)
    x1 = jnp.maximum((p_ref[0] + p_ref[1] + h_ref[...]) * dis + b_ref[...], 0.0)
    o_ref[...] = jnp.dot(x1, w_ref[...], preferred_element_type=jnp.float32) * dis


def _t3_body(deg_ref, p_ref, h_ref, b_ref, o_ref):
    o_ref[...] = (p_ref[0] + p_ref[1] + h_ref[...]) * _dis(deg_ref) + b_ref[...]


def _tc_h1(degp, xp, w1):
    return pl.pallas_call(
        _t1_body,
        grid=(GRID,),
        in_specs=[
            pl.BlockSpec((NC, BR), lambda i: (0, i)),
            pl.BlockSpec((BR, D), lambda i: (i, 0)),
            pl.BlockSpec((D, F1), lambda i: (0, 0)),
        ],
        out_specs=pl.BlockSpec((BR, F1), lambda i: (i, 0)),
        out_shape=jax.ShapeDtypeStruct((NP, F1), jnp.float32),
    )(degp, xp, w1)


def _tc_h2(degp, p1, h1, b1, w2p):
    return pl.pallas_call(
        _t2_body,
        grid=(GRID,),
        in_specs=[
            pl.BlockSpec((NC, BR), lambda i: (0, i)),
            pl.BlockSpec((NC, BR, F1), lambda i: (0, i, 0)),
            pl.BlockSpec((BR, F1), lambda i: (i, 0)),
            pl.BlockSpec((1, F1), lambda i: (0, 0)),
            pl.BlockSpec((F1, F2), lambda i: (0, 0)),
        ],
        out_specs=pl.BlockSpec((BR, F2), lambda i: (i, 0)),
        out_shape=jax.ShapeDtypeStruct((NP, F2), jnp.float32),
    )(degp, p1, h1, b1, w2p)


def _tc_out(degp, p2, h2, b2p):
    return pl.pallas_call(
        _t3_body,
        grid=(GRID,),
        in_specs=[
            pl.BlockSpec((NC, BR), lambda i: (0, i)),
            pl.BlockSpec((NC, BR, F2), lambda i: (0, i, 0)),
            pl.BlockSpec((BR, F2), lambda i: (i, 0)),
            pl.BlockSpec((1, F2), lambda i: (0, 0)),
        ],
        out_specs=pl.BlockSpec((BR, F2), lambda i: (i, 0)),
        out_shape=jax.ShapeDtypeStruct((NP, F2), jnp.float32),
    )(degp, p2, h2, b2p)


def kernel(features, edge_idx, W1, b1, W2, b2):
    row = edge_idx[0]
    col = edge_idx[1]
    pad = EPAD - E
    rowp = jnp.concatenate(
        [row, jnp.zeros((pad,), jnp.int32)]).reshape(NW, NCH, CH)
    colp = jnp.concatenate(
        [col, jnp.full((pad,), N, jnp.int32)]).reshape(NW, NCH, CH)

    xp = jnp.pad(features, ((0, NP - N), (0, 0)))
    w2p = jnp.pad(W2, ((0, 0), (0, F2 - C)))
    b1r = b1.reshape(1, F1)
    b2r = jnp.pad(b2, (0, F2 - C)).reshape(1, F2)
    z1 = jnp.zeros((NP,), jnp.float32)
    z64 = jnp.zeros((NP, F1), jnp.float32)
    z48 = jnp.zeros((NP, F2), jnp.float32)

    degp = _sc_degree(colp, z1)
    h1 = _tc_h1(degp, xp, W1)
    p1 = _sc_gather_add_f1(h1, rowp, colp, z64)
    h2 = _tc_h2(degp, p1, h1, b1r, w2p)
    p2 = _sc_gather_add_f2(h2, rowp, colp, z48)
    out = _tc_out(degp, p2, h2, b2r)
    return out[:N, :C]


# speedup vs baseline: 42.2901x; 1.6682x over previous
import functools

import jax
import jax.numpy as jnp
from jax import lax
from jax.experimental import pallas as pl
from jax.experimental.pallas import tpu as pltpu
from jax.experimental.pallas import tpu_sc as plsc

N = 10000
E = 320000
D = 128
F1 = 64
C = 40
F2 = 48

NP = 10240
NC = 2
NS = 16
NW = NC * NS
CH = 128
NCH = 79
EPT = NCH * CH
EPAD = NW * EPT
SLC = NP // NS
NG = 4
NSC = 4
NBUF = NG + NSC

BR = 512
GRID = NP // BR

_MESH = plsc.VectorSubcoreMesh(core_axis_name="c", subcore_axis_name="s")


@functools.partial(
    pl.kernel,
    out_type=jax.ShapeDtypeStruct((NC, NP), jnp.float32),
    mesh=_MESH,
    scratch_types=[
        pltpu.VMEM((NCH, CH), jnp.int32),
        pltpu.VMEM((CH,), jnp.float32),
        pltpu.VMEM_SHARED((NP,), jnp.float32),
    ],
)
def _sc_degree(col_hbm, zero1_hbm, out_hbm, colv, ones_v, accum):
    cid = lax.axis_index("c")
    sid = lax.axis_index("s")
    wid = sid * NC + cid
    for i in range(CH // 16):
        ones_v[pl.ds(i * 16, 16)] = jnp.ones((16,), jnp.float32)
    pltpu.sync_copy(zero1_hbm.at[pl.ds(sid * SLC, SLC)],
                    accum.at[pl.ds(sid * SLC, SLC)])
    pltpu.sync_copy(col_hbm.at[wid], colv)
    plsc.subcore_barrier()

    def body(ch, carry):
        pltpu.sync_copy(ones_v, accum.at[colv.at[ch]], add=True)
        return carry

    lax.fori_loop(0, NCH, body, 0)
    plsc.subcore_barrier()
    pltpu.sync_copy(accum.at[pl.ds(sid * SLC, SLC)],
                    out_hbm.at[cid, pl.ds(sid * SLC, SLC)])


def _make_sc_gather_add(F):
    @functools.partial(
        pl.kernel,
        out_type=jax.ShapeDtypeStruct((NC, NP, F), jnp.float32),
        mesh=_MESH,
        compiler_params=pltpu.CompilerParams(use_tc_tiling_on_sc=False),
        scratch_types=[
            pltpu.VMEM((NCH, CH), jnp.int32),
            pltpu.VMEM((NCH, CH), jnp.int32),
            pltpu.VMEM((NBUF, CH, F), jnp.float32),
            pltpu.VMEM_SHARED((NP, F), jnp.float32),
            pltpu.SemaphoreType.DMA((NBUF,)),
            pltpu.SemaphoreType.DMA((NBUF,)),
        ],
    )
    def sc_gather_add(table_hbm, row_hbm, col_hbm, zero2_hbm, out_hbm,
                      rowv, colv, rows_v, accum, gsem, ssem):
        cid = lax.axis_index("c")
        sid = lax.axis_index("s")
        wid = sid * NC + cid
        pltpu.sync_copy(zero2_hbm.at[pl.ds(sid * SLC, SLC)],
                        accum.at[pl.ds(sid * SLC, SLC)])
        pltpu.sync_copy(row_hbm.at[wid], rowv)
        pltpu.sync_copy(col_hbm.at[wid], colv)
        plsc.subcore_barrier()

        for b in range(NG):
            pltpu.async_copy(table_hbm.at[rowv.at[b]], rows_v.at[b],
                             gsem.at[b])

        def body(ch, carry):
            buf = lax.rem(ch, NBUF)
            pltpu.make_async_copy(table_hbm.at[rowv.at[ch]],
                                  rows_v.at[buf], gsem.at[buf]).wait()
            pltpu.async_copy(rows_v.at[buf], accum.at[colv.at[ch]],
                             ssem.at[buf], add=True)
            prv = ch - NSC
            nxt = ch + NG

            @pl.when(nxt < NCH)
            def _():
                pbuf = lax.rem(nxt, NBUF)

                @pl.when(prv >= 0)
                def _():
                    pltpu.make_async_copy(rows_v.at[pbuf],
                                          accum.at[colv.at[ch]],
                                          ssem.at[pbuf]).wait()

                pltpu.async_copy(table_hbm.at[rowv.at[nxt]],
                                 rows_v.at[pbuf], gsem.at[pbuf])

            return carry

        lax.fori_loop(0, NCH, body, 0)
        for k in range(max(0, NCH - NSC - NG), NCH):
            pltpu.make_async_copy(rows_v.at[k % NBUF],
                                  accum.at[colv.at[0]],
                                  ssem.at[k % NBUF]).wait()
        plsc.subcore_barrier()
        pltpu.sync_copy(accum.at[pl.ds(sid * SLC, SLC)],
                        out_hbm.at[cid, pl.ds(sid * SLC, SLC)])

    return sc_gather_add


_sc_gather_add_f1 = _make_sc_gather_add(F1)
_sc_gather_add_f2 = _make_sc_gather_add(F2)


def _dis(deg_ref):
    return lax.rsqrt(deg_ref[0] + deg_ref[1] + 1.0)[:, None]


def _t1_body(deg_ref, x_ref, w_ref, o_ref):
    m = jnp.dot(x_ref[...], w_ref[...], preferred_element_type=jnp.float32)
    o_ref[...] = m * _dis(deg_ref)


def _t2_body(deg_ref, p_ref, h_ref, b_ref, w_ref, o_ref):
    dis = _dis(deg_ref)
    x1 = jnp.maximum((p_ref[0] + p_ref[1] + h_ref[...]) * dis + b_ref[...], 0.0)
    o_ref[...] = jnp.dot(x1, w_ref[...], preferred_element_type=jnp.float32) * dis


def _t3_body(deg_ref, p_ref, h_ref, b_ref, o_ref):
    o_ref[...] = (p_ref[0] + p_ref[1] + h_ref[...]) * _dis(deg_ref) + b_ref[...]


def _tc_h1(degp, xp, w1):
    return pl.pallas_call(
        _t1_body,
        grid=(GRID,),
        in_specs=[
            pl.BlockSpec((NC, BR), lambda i: (0, i)),
            pl.BlockSpec((BR, D), lambda i: (i, 0)),
            pl.BlockSpec((D, F1), lambda i: (0, 0)),
        ],
        out_specs=pl.BlockSpec((BR, F1), lambda i: (i, 0)),
        out_shape=jax.ShapeDtypeStruct((NP, F1), jnp.float32),
    )(degp, xp, w1)


def _tc_h2(degp, p1, h1, b1, w2p):
    return pl.pallas_call(
        _t2_body,
        grid=(GRID,),
        in_specs=[
            pl.BlockSpec((NC, BR), lambda i: (0, i)),
            pl.BlockSpec((NC, BR, F1), lambda i: (0, i, 0)),
            pl.BlockSpec((BR, F1), lambda i: (i, 0)),
            pl.BlockSpec((1, F1), lambda i: (0, 0)),
            pl.BlockSpec((F1, F2), lambda i: (0, 0)),
        ],
        out_specs=pl.BlockSpec((BR, F2), lambda i: (i, 0)),
        out_shape=jax.ShapeDtypeStruct((NP, F2), jnp.float32),
    )(degp, p1, h1, b1, w2p)


def _tc_out(degp, p2, h2, b2p):
    return pl.pallas_call(
        _t3_body,
        grid=(GRID,),
        in_specs=[
            pl.BlockSpec((NC, BR), lambda i: (0, i)),
            pl.BlockSpec((NC, BR, F2), lambda i: (0, i, 0)),
            pl.BlockSpec((BR, F2), lambda i: (i, 0)),
            pl.BlockSpec((1, F2), lambda i: (0, 0)),
        ],
        out_specs=pl.BlockSpec((BR, F2), lambda i: (i, 0)),
        out_shape=jax.ShapeDtypeStruct((NP, F2), jnp.float32),
    )(degp, p2, h2, b2p)


def kernel(features, edge_idx, W1, b1, W2, b2):
    row = edge_idx[0]
    col = edge_idx[1]
    ppw = EPT - E // NW
    dum = jnp.arange(ppw, dtype=jnp.int32)
    rowp = jnp.concatenate(
        [row.reshape(NW, E // NW),
         jnp.broadcast_to(dum, (NW, ppw))], axis=1).reshape(NW, NCH, CH)
    colp = jnp.concatenate(
        [col.reshape(NW, E // NW),
         jnp.broadcast_to(N + dum, (NW, ppw))], axis=1).reshape(NW, NCH, CH)

    xp = jnp.pad(features, ((0, NP - N), (0, 0)))
    w2p = jnp.pad(W2, ((0, 0), (0, F2 - C)))
    b1r = b1.reshape(1, F1)
    b2r = jnp.pad(b2, (0, F2 - C)).reshape(1, F2)
    z1 = jnp.zeros((NP,), jnp.float32)
    z64 = jnp.zeros((NP, F1), jnp.float32)
    z48 = jnp.zeros((NP, F2), jnp.float32)

    degp = _sc_degree(colp, z1)
    h1 = _tc_h1(degp, xp, W1)
    p1 = _sc_gather_add_f1(h1, rowp, colp, z64)
    h2 = _tc_h2(degp, p1, h1, b1r, w2p)
    p2 = _sc_gather_add_f2(h2, rowp, colp, z48)
    out = _tc_out(degp, p2, h2, b2r)
    return out[:N, :C]


# speedup vs baseline: 47.1683x; 1.1154x over previous
import functools

import jax
import jax.numpy as jnp
from jax import lax
from jax.experimental import pallas as pl
from jax.experimental.pallas import tpu as pltpu
from jax.experimental.pallas import tpu_sc as plsc

N = 10000
E = 320000
D = 128
F1 = 64
C = 40
F2 = 48

NP = 10240
NC = 2
NS = 16
NW = NC * NS
CH = 128
NCH = 79
EPT = NCH * CH
EPAD = NW * EPT
SLC = NP // NS
RING = {F1: (4, 4), F2: (6, 6)}

BR = 2048
GRID = NP // BR

_MESH = plsc.VectorSubcoreMesh(core_axis_name="c", subcore_axis_name="s")


@functools.partial(
    pl.kernel,
    out_type=jax.ShapeDtypeStruct((NC, NP), jnp.float32),
    mesh=_MESH,
    scratch_types=[
        pltpu.VMEM((NCH, CH), jnp.int32),
        pltpu.VMEM((CH,), jnp.float32),
        pltpu.VMEM_SHARED((NP,), jnp.float32),
    ],
)
def _sc_degree(col_hbm, zero1_hbm, out_hbm, colv, ones_v, accum):
    cid = lax.axis_index("c")
    sid = lax.axis_index("s")
    wid = sid * NC + cid
    for i in range(CH // 16):
        ones_v[pl.ds(i * 16, 16)] = jnp.ones((16,), jnp.float32)
    pltpu.sync_copy(zero1_hbm.at[pl.ds(sid * SLC, SLC)],
                    accum.at[pl.ds(sid * SLC, SLC)])
    pltpu.sync_copy(col_hbm.at[wid], colv)
    plsc.subcore_barrier()

    def body(ch, carry):
        pltpu.sync_copy(ones_v, accum.at[colv.at[ch]], add=True)
        return carry

    lax.fori_loop(0, NCH, body, 0)
    plsc.subcore_barrier()
    pltpu.sync_copy(accum.at[pl.ds(sid * SLC, SLC)],
                    out_hbm.at[cid, pl.ds(sid * SLC, SLC)])


def _make_sc_gather_add(F):
    NG, NSC = RING[F]
    NBUF = NG + NSC

    @functools.partial(
        pl.kernel,
        out_type=jax.ShapeDtypeStruct((NC, NP, F), jnp.float32),
        mesh=_MESH,
        compiler_params=pltpu.CompilerParams(use_tc_tiling_on_sc=False),
        scratch_types=[
            pltpu.VMEM((NCH, CH), jnp.int32),
            pltpu.VMEM((NCH, CH), jnp.int32),
            pltpu.VMEM((NBUF, CH, F), jnp.float32),
            pltpu.VMEM_SHARED((NP, F), jnp.float32),
            pltpu.SemaphoreType.DMA((NBUF,)),
            pltpu.SemaphoreType.DMA((NBUF,)),
        ],
    )
    def sc_gather_add(table_hbm, row_hbm, col_hbm, zero2_hbm, out_hbm,
                      rowv, colv, rows_v, accum, gsem, ssem):
        cid = lax.axis_index("c")
        sid = lax.axis_index("s")
        wid = sid * NC + cid
        pltpu.sync_copy(zero2_hbm.at[pl.ds(sid * SLC, SLC)],
                        accum.at[pl.ds(sid * SLC, SLC)])
        pltpu.sync_copy(row_hbm.at[wid], rowv)
        pltpu.sync_copy(col_hbm.at[wid], colv)
        plsc.subcore_barrier()

        for b in range(NG):
            pltpu.async_copy(table_hbm.at[rowv.at[b]], rows_v.at[b],
                             gsem.at[b])

        def body(ch, carry):
            buf = lax.rem(ch, NBUF)
            pltpu.make_async_copy(table_hbm.at[rowv.at[ch]],
                                  rows_v.at[buf], gsem.at[buf]).wait()
            pltpu.async_copy(rows_v.at[buf], accum.at[colv.at[ch]],
                             ssem.at[buf], add=True)
            prv = ch - NSC
            nxt = ch + NG

            @pl.when(nxt < NCH)
            def _():
                pbuf = lax.rem(nxt, NBUF)

                @pl.when(prv >= 0)
                def _():
                    pltpu.make_async_copy(rows_v.at[pbuf],
                                          accum.at[colv.at[ch]],
                                          ssem.at[pbuf]).wait()

                pltpu.async_copy(table_hbm.at[rowv.at[nxt]],
                                 rows_v.at[pbuf], gsem.at[pbuf])

            return carry

        lax.fori_loop(0, NCH, body, 0)
        for k in range(max(0, NCH - NSC - NG), NCH):
            pltpu.make_async_copy(rows_v.at[k % NBUF],
                                  accum.at[colv.at[0]],
                                  ssem.at[k % NBUF]).wait()
        plsc.subcore_barrier()
        pltpu.sync_copy(accum.at[pl.ds(sid * SLC, SLC)],
                        out_hbm.at[cid, pl.ds(sid * SLC, SLC)])

    return sc_gather_add


_sc_gather_add_f1 = _make_sc_gather_add(F1)
_sc_gather_add_f2 = _make_sc_gather_add(F2)


def _dis(deg_ref):
    return lax.rsqrt(deg_ref[0] + deg_ref[1] + 1.0)[:, None]


def _t1_body(deg_ref, x_ref, w_ref, o_ref):
    m = jnp.dot(x_ref[...], w_ref[...], preferred_element_type=jnp.float32)
    o_ref[...] = m * _dis(deg_ref)


def _t2_body(deg_ref, p_ref, h_ref, b_ref, w_ref, o_ref):
    dis = _dis(deg_ref)
    x1 = jnp.maximum((p_ref[0] + p_ref[1] + h_ref[...]) * dis + b_ref[...], 0.0)
    o_ref[...] = jnp.dot(x1, w_ref[...], preferred_element_type=jnp.float32) * dis


def _t3_body(deg_ref, p_ref, h_ref, b_ref, o_ref):
    o_ref[...] = (p_ref[0] + p_ref[1] + h_ref[...]) * _dis(deg_ref) + b_ref[...]


def _tc_h1(degp, xp, w1):
    return pl.pallas_call(
        _t1_body,
        grid=(GRID,),
        in_specs=[
            pl.BlockSpec((NC, BR), lambda i: (0, i)),
            pl.BlockSpec((BR, D), lambda i: (i, 0)),
            pl.BlockSpec((D, F1), lambda i: (0, 0)),
        ],
        out_specs=pl.BlockSpec((BR, F1), lambda i: (i, 0)),
        out_shape=jax.ShapeDtypeStruct((NP, F1), jnp.float32),
    )(degp, xp, w1)


def _tc_h2(degp, p1, h1, b1, w2p):
    return pl.pallas_call(
        _t2_body,
        grid=(GRID,),
        in_specs=[
            pl.BlockSpec((NC, BR), lambda i: (0, i)),
            pl.BlockSpec((NC, BR, F1), lambda i: (0, i, 0)),
            pl.BlockSpec((BR, F1), lambda i: (i, 0)),
            pl.BlockSpec((1, F1), lambda i: (0, 0)),
            pl.BlockSpec((F1, F2), lambda i: (0, 0)),
        ],
        out_specs=pl.BlockSpec((BR, F2), lambda i: (i, 0)),
        out_shape=jax.ShapeDtypeStruct((NP, F2), jnp.float32),
    )(degp, p1, h1, b1, w2p)


def _tc_out(degp, p2, h2, b2p):
    return pl.pallas_call(
        _t3_body,
        grid=(GRID,),
        in_specs=[
            pl.BlockSpec((NC, BR), lambda i: (0, i)),
            pl.BlockSpec((NC, BR, F2), lambda i: (0, i, 0)),
            pl.BlockSpec((BR, F2), lambda i: (i, 0)),
            pl.BlockSpec((1, F2), lambda i: (0, 0)),
        ],
        out_specs=pl.BlockSpec((BR, F2), lambda i: (i, 0)),
        out_shape=jax.ShapeDtypeStruct((NP, F2), jnp.float32),
    )(degp, p2, h2, b2p)


def kernel(features, edge_idx, W1, b1, W2, b2):
    ppw = EPT - E // NW
    dum = jnp.arange(ppw, dtype=jnp.int32)
    rowp = jnp.concatenate(
        [edge_idx[0].reshape(NW, E // NW),
         jnp.broadcast_to(dum, (NW, ppw))], axis=1).reshape(NW, NCH, CH)
    colp = jnp.concatenate(
        [edge_idx[1].reshape(NW, E // NW),
         jnp.broadcast_to(N + dum, (NW, ppw))], axis=1).reshape(NW, NCH, CH)

    xp = jnp.pad(features, ((0, NP - N), (0, 0)))
    w2p = jnp.pad(W2, ((0, 0), (0, F2 - C)))
    b1r = b1.reshape(1, F1)
    b2r = jnp.pad(b2, (0, F2 - C)).reshape(1, F2)
    z1 = jnp.zeros((NP,), jnp.float32)
    z64 = jnp.zeros((NP, F1), jnp.float32)
    z48 = jnp.zeros((NP, F2), jnp.float32)

    degp = _sc_degree(colp, z1)
    h1 = _tc_h1(degp, xp, W1)
    p1 = _sc_gather_add_f1(h1, rowp, colp, z64)
    h2 = _tc_h2(degp, p1, h1, b1r, w2p)
    p2 = _sc_gather_add_f2(h2, rowp, colp, z48)
    out = _tc_out(degp, p2, h2, b2r)
    return out[:N, :C]


# speedup vs baseline: 48.8345x; 1.0353x over previous
import functools

import jax
import jax.numpy as jnp
from jax import lax
from jax.experimental import pallas as pl
from jax.experimental.pallas import tpu as pltpu
from jax.experimental.pallas import tpu_sc as plsc

N = 10000
E = 320000
D = 128
F1 = 64
C = 40
F2 = 48

NP = 10240
NC = 2
NS = 16
NW = NC * NS
CH = 128
NCH = 79
EPT = NCH * CH
EPAD = NW * EPT
SLC = NP // NS
RING = {F1: (4, 4), F2: (6, 6)}

BR = 2048
GRID = NP // BR

_MESH = plsc.VectorSubcoreMesh(core_axis_name="c", subcore_axis_name="s")


@functools.partial(
    pl.kernel,
    out_type=jax.ShapeDtypeStruct((NC, NP), jnp.float32),
    mesh=_MESH,
    scratch_types=[
        pltpu.VMEM((NCH, CH), jnp.int32),
        pltpu.VMEM((CH,), jnp.float32),
        pltpu.VMEM_SHARED((NP,), jnp.float32),
    ],
)
def _sc_degree(col_hbm, zero1_hbm, out_hbm, colv, ones_v, accum):
    cid = lax.axis_index("c")
    sid = lax.axis_index("s")
    wid = sid * NC + cid
    for i in range(CH // 16):
        ones_v[pl.ds(i * 16, 16)] = jnp.ones((16,), jnp.float32)
    pltpu.sync_copy(zero1_hbm.at[pl.ds(sid * SLC, SLC)],
                    accum.at[pl.ds(sid * SLC, SLC)])
    pltpu.sync_copy(col_hbm.at[wid], colv)
    plsc.subcore_barrier()

    def body(ch, carry):
        pltpu.sync_copy(ones_v, accum.at[colv.at[ch]], add=True)
        return carry

    lax.fori_loop(0, NCH, body, 0)
    plsc.subcore_barrier()
    pltpu.sync_copy(accum.at[pl.ds(sid * SLC, SLC)],
                    out_hbm.at[cid, pl.ds(sid * SLC, SLC)])


def _make_sc_gather_add(F):
    NG, NSC = RING[F]
    NBUF = NG + NSC

    @functools.partial(
        pl.kernel,
        out_type=jax.ShapeDtypeStruct((NC, NP, F), jnp.float32),
        mesh=_MESH,
        compiler_params=pltpu.CompilerParams(use_tc_tiling_on_sc=False),
        scratch_types=[
            pltpu.VMEM((NCH, CH), jnp.int32),
            pltpu.VMEM((NCH, CH), jnp.int32),
            pltpu.VMEM((NBUF, CH, F), jnp.float32),
            pltpu.VMEM_SHARED((NP, F), jnp.float32),
            pltpu.SemaphoreType.DMA((NBUF,)),
            pltpu.SemaphoreType.DMA((NBUF,)),
        ],
    )
    def sc_gather_add(table_hbm, row_hbm, col_hbm, zero2_hbm, out_hbm,
                      rowv, colv, rows_v, accum, gsem, ssem):
        cid = lax.axis_index("c")
        sid = lax.axis_index("s")
        wid = sid * NC + cid
        pltpu.sync_copy(zero2_hbm.at[pl.ds(sid * SLC, SLC)],
                        accum.at[pl.ds(sid * SLC, SLC)])
        pltpu.sync_copy(row_hbm.at[wid], rowv)
        pltpu.sync_copy(col_hbm.at[wid], colv)
        plsc.subcore_barrier()

        for b in range(NG):
            pltpu.async_copy(table_hbm.at[rowv.at[b]], rows_v.at[b],
                             gsem.at[b])

        def body(ch, carry):
            buf = lax.rem(ch, NBUF)
            pltpu.make_async_copy(table_hbm.at[rowv.at[ch]],
                                  rows_v.at[buf], gsem.at[buf]).wait()
            pltpu.async_copy(rows_v.at[buf], accum.at[colv.at[ch]],
                             ssem.at[buf], add=True)
            prv = ch - NSC
            nxt = ch + NG

            @pl.when(nxt < NCH)
            def _():
                pbuf = lax.rem(nxt, NBUF)

                @pl.when(prv >= 0)
                def _():
                    pltpu.make_async_copy(rows_v.at[pbuf],
                                          accum.at[colv.at[ch]],
                                          ssem.at[pbuf]).wait()

                pltpu.async_copy(table_hbm.at[rowv.at[nxt]],
                                 rows_v.at[pbuf], gsem.at[pbuf])

            return carry

        lax.fori_loop(0, NCH, body, 0)
        for k in range(max(0, NCH - NSC - NG), NCH):
            pltpu.make_async_copy(rows_v.at[k % NBUF],
                                  accum.at[colv.at[0]],
                                  ssem.at[k % NBUF]).wait()
        plsc.subcore_barrier()
        pltpu.sync_copy(accum.at[pl.ds(sid * SLC, SLC)],
                        out_hbm.at[cid, pl.ds(sid * SLC, SLC)])

    return sc_gather_add


_sc_gather_add_f1 = _make_sc_gather_add(F1)
_sc_gather_add_f2 = _make_sc_gather_add(F2)


def _dis(deg_ref):
    return lax.rsqrt(deg_ref[0] + deg_ref[1] + 1.0)[:, None]


def _t1_body(deg_ref, x_ref, w_ref, o_ref):
    m = jnp.dot(x_ref[...], w_ref[...], preferred_element_type=jnp.float32)
    o_ref[...] = m * _dis(deg_ref)


def _t2_body(deg_ref, p_ref, h_ref, b_ref, w_ref, o_ref):
    dis = _dis(deg_ref)
    x1 = jnp.maximum((p_ref[0] + p_ref[1] + h_ref[...]) * dis + b_ref[...], 0.0)
    o_ref[...] = jnp.dot(x1, w_ref[...], preferred_element_type=jnp.float32) * dis


def _t3_body(deg_ref, p_ref, h_ref, b_ref, o_ref):
    v = (p_ref[0] + p_ref[1] + h_ref[...]) * _dis(deg_ref) + b_ref[...]
    o_ref[...] = v[:, :C]


def _tc_h1(degp, xp, w1):
    return pl.pallas_call(
        _t1_body,
        grid=(GRID,),
        in_specs=[
            pl.BlockSpec((NC, BR), lambda i: (0, i)),
            pl.BlockSpec((BR, D), lambda i: (i, 0)),
            pl.BlockSpec((D, F1), lambda i: (0, 0)),
        ],
        out_specs=pl.BlockSpec((BR, F1), lambda i: (i, 0)),
        out_shape=jax.ShapeDtypeStruct((NP, F1), jnp.float32),
    )(degp, xp, w1)


def _tc_h2(degp, p1, h1, b1, w2p):
    return pl.pallas_call(
        _t2_body,
        grid=(GRID,),
        in_specs=[
            pl.BlockSpec((NC, BR), lambda i: (0, i)),
            pl.BlockSpec((NC, BR, F1), lambda i: (0, i, 0)),
            pl.BlockSpec((BR, F1), lambda i: (i, 0)),
            pl.BlockSpec((1, F1), lambda i: (0, 0)),
            pl.BlockSpec((F1, F2), lambda i: (0, 0)),
        ],
        out_specs=pl.BlockSpec((BR, F2), lambda i: (i, 0)),
        out_shape=jax.ShapeDtypeStruct((NP, F2), jnp.float32),
    )(degp, p1, h1, b1, w2p)


def _tc_out(degp, p2, h2, b2p):
    return pl.pallas_call(
        _t3_body,
        grid=(GRID,),
        in_specs=[
            pl.BlockSpec((NC, BR), lambda i: (0, i)),
            pl.BlockSpec((NC, BR, F2), lambda i: (0, i, 0)),
            pl.BlockSpec((BR, F2), lambda i: (i, 0)),
            pl.BlockSpec((1, F2), lambda i: (0, 0)),
        ],
        out_specs=pl.BlockSpec((BR, C), lambda i: (i, 0)),
        out_shape=jax.ShapeDtypeStruct((N, C), jnp.float32),
    )(degp, p2, h2, b2p)


def kernel(features, edge_idx, W1, b1, W2, b2):
    ppw = EPT - E // NW
    dum = jnp.arange(ppw, dtype=jnp.int32)
    rowp = jnp.concatenate(
        [edge_idx[0].reshape(NW, E // NW),
         jnp.broadcast_to(dum, (NW, ppw))], axis=1).reshape(NW, NCH, CH)
    colp = jnp.concatenate(
        [edge_idx[1].reshape(NW, E // NW),
         jnp.broadcast_to(N + dum, (NW, ppw))], axis=1).reshape(NW, NCH, CH)

    xp = jnp.pad(features, ((0, NP - N), (0, 0)))
    w2p = jnp.pad(W2, ((0, 0), (0, F2 - C)))
    b1r = b1.reshape(1, F1)
    b2r = jnp.pad(b2, (0, F2 - C)).reshape(1, F2)
    z1 = jnp.zeros((NP,), jnp.float32)
    z64 = jnp.zeros((NP, F1), jnp.float32)
    z48 = jnp.zeros((NP, F2), jnp.float32)

    degp = _sc_degree(colp, z1)
    h1 = _tc_h1(degp, xp, W1)
    p1 = _sc_gather_add_f1(h1, rowp, colp, z64)
    h2 = _tc_h2(degp, p1, h1, b1r, w2p)
    p2 = _sc_gather_add_f2(h2, rowp, colp, z48)
    return _tc_out(degp, p2, h2, b2r)


# speedup vs baseline: 48.8711x; 1.0007x over previous
import functools

import jax
import jax.numpy as jnp
from jax import lax
from jax.experimental import pallas as pl
from jax.experimental.pallas import tpu as pltpu
from jax.experimental.pallas import tpu_sc as plsc

N = 10000
E = 320000
D = 128
F1 = 64
C = 40
F2 = 48

NP = 10240
NC = 2
NS = 16
NW = NC * NS
CH = 128
TCH = E // CH
TCHP = 2504
NCH = 88
SLC = NP // NS
RING = {F1: (4, 4), F2: (6, 6)}

BR = 2048
GRID = NP // BR

_MESH = plsc.VectorSubcoreMesh(core_axis_name="c", subcore_axis_name="s")


@functools.partial(
    pl.kernel,
    out_type=jax.ShapeDtypeStruct((NC, NP), jnp.float32),
    mesh=_MESH,
    compiler_params=pltpu.CompilerParams(use_tc_tiling_on_sc=False),
    scratch_types=[
        pltpu.VMEM((NCH, CH), jnp.int32),
        pltpu.VMEM((CH,), jnp.float32),
        pltpu.VMEM_SHARED((NP,), jnp.float32),
    ],
)
def _sc_degree(col_hbm, zero1_hbm, out_hbm, colv, ones_v, accum):
    cid = lax.axis_index("c")
    sid = lax.axis_index("s")
    wid = sid * NC + cid
    start = ((wid * TCH) // (NW * 8)) * 8
    nxt_s = (((wid + 1) * TCH) // (NW * 8)) * 8
    cnt = jnp.where(wid == NW - 1, TCH, nxt_s) - start
    for i in range(CH // 16):
        ones_v[pl.ds(i * 16, 16)] = jnp.ones((16,), jnp.float32)
    pltpu.sync_copy(zero1_hbm.at[pl.ds(sid * SLC, SLC)],
                    accum.at[pl.ds(sid * SLC, SLC)])
    pltpu.sync_copy(col_hbm.at[pl.ds(start, NCH)], colv)
    plsc.subcore_barrier()

    def body(ch, carry):
        pltpu.sync_copy(ones_v, accum.at[colv.at[ch]], add=True)
        return carry

    lax.fori_loop(0, cnt, body, 0)
    plsc.subcore_barrier()
    pltpu.sync_copy(accum.at[pl.ds(sid * SLC, SLC)],
                    out_hbm.at[cid, pl.ds(sid * SLC, SLC)])


def _make_sc_gather_add(F):
    NG, NSC = RING[F]
    NBUF = NG + NSC

    @functools.partial(
        pl.kernel,
        out_type=jax.ShapeDtypeStruct((NC, NP, F), jnp.float32),
        mesh=_MESH,
        compiler_params=pltpu.CompilerParams(use_tc_tiling_on_sc=False),
        scratch_types=[
            pltpu.VMEM((NCH, CH), jnp.int32),
            pltpu.VMEM((NCH, CH), jnp.int32),
            pltpu.VMEM((NBUF, CH, F), jnp.float32),
            pltpu.VMEM_SHARED((NP, F), jnp.float32),
            pltpu.SemaphoreType.DMA((NBUF,)),
            pltpu.SemaphoreType.DMA((NBUF,)),
        ],
    )
    def sc_gather_add(table_hbm, row_hbm, col_hbm, zero2_hbm, out_hbm,
                      rowv, colv, rows_v, accum, gsem, ssem):
        cid = lax.axis_index("c")
        sid = lax.axis_index("s")
        wid = sid * NC + cid
        start = ((wid * TCH) // (NW * 8)) * 8
        nxt_s = (((wid + 1) * TCH) // (NW * 8)) * 8
        cnt = jnp.where(wid == NW - 1, TCH, nxt_s) - start
        pltpu.sync_copy(zero2_hbm.at[pl.ds(sid * SLC, SLC)],
                        accum.at[pl.ds(sid * SLC, SLC)])
        pltpu.sync_copy(row_hbm.at[pl.ds(start, NCH)], rowv)
        pltpu.sync_copy(col_hbm.at[pl.ds(start, NCH)], colv)
        plsc.subcore_barrier()

        for b in range(NG):
            pltpu.async_copy(table_hbm.at[rowv.at[b]], rows_v.at[b],
                             gsem.at[b])

        def body(ch, carry):
            buf = lax.rem(ch, NBUF)
            pltpu.make_async_copy(table_hbm.at[rowv.at[ch]],
                                  rows_v.at[buf], gsem.at[buf]).wait()
            pltpu.async_copy(rows_v.at[buf], accum.at[colv.at[ch]],
                             ssem.at[buf], add=True)
            prv = ch - NSC
            nxt = ch + NG

            @pl.when(nxt < cnt)
            def _():
                pbuf = lax.rem(nxt, NBUF)

                @pl.when(prv >= 0)
                def _():
                    pltpu.make_async_copy(rows_v.at[pbuf],
                                          accum.at[colv.at[ch]],
                                          ssem.at[pbuf]).wait()

                pltpu.async_copy(table_hbm.at[rowv.at[nxt]],
                                 rows_v.at[pbuf], gsem.at[pbuf])

            return carry

        lax.fori_loop(0, cnt, body, 0)

        def drain(k, carry):
            pltpu.make_async_copy(rows_v.at[lax.rem(k, NBUF)],
                                  accum.at[colv.at[0]],
                                  ssem.at[lax.rem(k, NBUF)]).wait()
            return carry

        lax.fori_loop(jnp.maximum(0, cnt - NSC - NG), cnt, drain, 0)
        plsc.subcore_barrier()
        pltpu.sync_copy(accum.at[pl.ds(sid * SLC, SLC)],
                        out_hbm.at[cid, pl.ds(sid * SLC, SLC)])

    return sc_gather_add


_sc_gather_add_f1 = _make_sc_gather_add(F1)
_sc_gather_add_f2 = _make_sc_gather_add(F2)


def _dis(deg_ref):
    return lax.rsqrt(deg_ref[0] + deg_ref[1] + 1.0)[:, None]


def _t1_body(deg_ref, x_ref, w_ref, o_ref):
    m = jnp.dot(x_ref[...], w_ref[...], preferred_element_type=jnp.float32)
    o_ref[...] = m * _dis(deg_ref)


def _t2_body(deg_ref, p_ref, h_ref, b_ref, w_ref, o_ref):
    dis = _dis(deg_ref)
    x1 = jnp.maximum((p_ref[0] + p_ref[1] + h_ref[...]) * dis + b_ref[...], 0.0)
    o_ref[...] = jnp.dot(x1, w_ref[...], preferred_element_type=jnp.float32) * dis


def _t3_body(deg_ref, p_ref, h_ref, b_ref, o_ref):
    v = (p_ref[0] + p_ref[1] + h_ref[...]) * _dis(deg_ref) + b_ref[...]
    o_ref[...] = v[:, :C]


def _tc_h1(degp, xp, w1):
    return pl.pallas_call(
        _t1_body,
        grid=(GRID,),
        in_specs=[
            pl.BlockSpec((NC, BR), lambda i: (0, i)),
            pl.BlockSpec((BR, D), lambda i: (i, 0)),
            pl.BlockSpec((D, F1), lambda i: (0, 0)),
        ],
        out_specs=pl.BlockSpec((BR, F1), lambda i: (i, 0)),
        out_shape=jax.ShapeDtypeStruct((NP, F1), jnp.float32),
    )(degp, xp, w1)


def _tc_h2(degp, p1, h1, b1, w2p):
    return pl.pallas_call(
        _t2_body,
        grid=(GRID,),
        in_specs=[
            pl.BlockSpec((NC, BR), lambda i: (0, i)),
            pl.BlockSpec((NC, BR, F1), lambda i: (0, i, 0)),
            pl.BlockSpec((BR, F1), lambda i: (i, 0)),
            pl.BlockSpec((1, F1), lambda i: (0, 0)),
            pl.BlockSpec((F1, F2), lambda i: (0, 0)),
        ],
        out_specs=pl.BlockSpec((BR, F2), lambda i: (i, 0)),
        out_shape=jax.ShapeDtypeStruct((NP, F2), jnp.float32),
    )(degp, p1, h1, b1, w2p)


def _tc_out(degp, p2, h2, b2p):
    return pl.pallas_call(
        _t3_body,
        grid=(GRID,),
        in_specs=[
            pl.BlockSpec((NC, BR), lambda i: (0, i)),
            pl.BlockSpec((NC, BR, F2), lambda i: (0, i, 0)),
            pl.BlockSpec((BR, F2), lambda i: (i, 0)),
            pl.BlockSpec((1, F2), lambda i: (0, 0)),
        ],
        out_specs=pl.BlockSpec((BR, C), lambda i: (i, 0)),
        out_shape=jax.ShapeDtypeStruct((N, C), jnp.float32),
    )(degp, p2, h2, b2p)


def _dt_body(x_ref, o_ref):
    xr = x_ref[...].reshape(2, TCH, CH)
    o_ref[...] = jnp.concatenate(
        [xr, jnp.zeros((2, TCHP - TCH, CH), jnp.int32)], axis=1)


def _tc_detile(edge_idx):
    return pl.pallas_call(
        _dt_body,
        out_shape=jax.ShapeDtypeStruct((2, TCHP, CH), jnp.int32),
    )(edge_idx)


def kernel(features, edge_idx, W1, b1, W2, b2):
    ec = _tc_detile(edge_idx)
    rowp = ec[0]
    colp = ec[1]

    xp = jnp.pad(features, ((0, NP - N), (0, 0)))
    w2p = jnp.pad(W2, ((0, 0), (0, F2 - C)))
    b1r = b1.reshape(1, F1)
    b2r = jnp.pad(b2, (0, F2 - C)).reshape(1, F2)
    z1 = jnp.zeros((NP,), jnp.float32)
    z64 = jnp.zeros((NP, F1), jnp.float32)
    z48 = jnp.zeros((NP, F2), jnp.float32)

    degp = _sc_degree(colp, z1)
    h1 = _tc_h1(degp, xp, W1)
    p1 = _sc_gather_add_f1(h1, rowp, colp, z64)
    h2 = _tc_h2(degp, p1, h1, b1r, w2p)
    p2 = _sc_gather_add_f2(h2, rowp, colp, z48)
    return _tc_out(degp, p2, h2, b2r)
